# baseline (speedup 1.0000x reference)
import functools

import jax
import jax.numpy as jnp
from jax import lax
from jax.experimental import pallas as pl
from jax.experimental.pallas import tpu as pltpu
from jax.experimental.pallas import tpu_sc as plsc

DIM = 128
CHUNK = 128
NC, NS = 2, 16
NW = NC * NS


@functools.lru_cache(maxsize=None)
def _gather_fn(n_per_w: int):
    mesh = plsc.VectorSubcoreMesh(core_axis_name="c", subcore_axis_name="s")
    NBUF = 5
    LOOK = 3

    @functools.partial(
        pl.kernel,
        mesh=mesh,
        out_type=jax.ShapeDtypeStruct((NW * n_per_w * CHUNK, DIM), jnp.float32),
        compiler_params=pltpu.CompilerParams(use_tc_tiling_on_sc=True),
        scratch_types=[pltpu.VMEM((n_per_w * CHUNK,), jnp.int32)]
        + [pltpu.VMEM((CHUNK, DIM), jnp.float32)] * NBUF
        + [pltpu.SemaphoreType.DMA] * (2 * NBUF),
    )
    def k(idx_hbm, table_hbm, out_hbm, idx_v, *bufs):
        rows = bufs[:NBUF]
        gs = bufs[NBUF:2 * NBUF]
        ws = bufs[2 * NBUF:]
        wid = lax.axis_index("s") * NC + lax.axis_index("c")
        cbase = wid * n_per_w
        pltpu.sync_copy(idx_hbm.at[pl.ds(cbase * CHUNK, n_per_w * CHUNK)], idx_v)

        def gather_start(j, b):
            pltpu.async_copy(
                table_hbm.at[idx_v.at[pl.ds(j * CHUNK, CHUNK)]], rows[b], gs[b])

        def gather_wait(j, b):
            pltpu.make_async_copy(
                table_hbm.at[idx_v.at[pl.ds(j * CHUNK, CHUNK)]], rows[b],
                gs[b]).wait()

        def write_start(j, b):
            pltpu.async_copy(
                rows[b], out_hbm.at[pl.ds((cbase + j) * CHUNK, CHUNK)], ws[b])

        def write_wait(j, b):
            pltpu.make_async_copy(
                rows[b], out_hbm.at[pl.ds((cbase + j) * CHUNK, CHUNK)],
                ws[b]).wait()

        for b in range(LOOK):
            gather_start(b, b)

        def body(g, carry):
            for b in range(NBUF):
                j = g * NBUF + b
                bp = (b + LOOK) % NBUF
                gather_wait(j, b)
                write_start(j, b)

                @pl.when(j + LOOK < n_per_w)
                def _():
                    @pl.when(j + LOOK >= NBUF)
                    def _w():
                        write_wait(j + LOOK - NBUF, bp)

                    gather_start(j + LOOK, bp)

            return carry

        lax.fori_loop(0, n_per_w // NBUF, body, 0)

        for b in range(NBUF):
            write_wait(n_per_w - NBUF + b, b)

    return k


def kernel(input, table):
    batch, hist = input.shape
    flat_t = jnp.transpose(input).reshape(-1).astype(jnp.int32)
    n_chunks = flat_t.shape[0] // CHUNK
    out = _gather_fn(n_chunks // NW)(flat_t, table)
    out = out.reshape(hist, batch, DIM)
    return jnp.transpose(out, (1, 0, 2))


# speedup vs baseline: 1.0032x; 1.0032x over previous
import functools

import jax
import jax.numpy as jnp
from jax import lax
from jax.experimental import pallas as pl
from jax.experimental.pallas import tpu as pltpu
from jax.experimental.pallas import tpu_sc as plsc

DIM = 128
CHUNK = 64
NC, NS = 2, 16
NW = NC * NS


@functools.lru_cache(maxsize=None)
def _gather_fn(n_per_w: int):
    mesh = plsc.VectorSubcoreMesh(core_axis_name="c", subcore_axis_name="s")
    NBUF = 8
    LOOK = 4

    @functools.partial(
        pl.kernel,
        mesh=mesh,
        out_type=jax.ShapeDtypeStruct((NW * n_per_w * CHUNK, DIM), jnp.float32),
        compiler_params=pltpu.CompilerParams(use_tc_tiling_on_sc=True),
        scratch_types=[pltpu.VMEM((n_per_w * CHUNK,), jnp.int32)]
        + [pltpu.VMEM((CHUNK, DIM), jnp.float32)] * NBUF
        + [pltpu.SemaphoreType.DMA] * (2 * NBUF),
    )
    def k(idx_hbm, table_hbm, out_hbm, idx_v, *bufs):
        rows = bufs[:NBUF]
        gs = bufs[NBUF:2 * NBUF]
        ws = bufs[2 * NBUF:]
        wid = lax.axis_index("s") * NC + lax.axis_index("c")
        cbase = wid * n_per_w
        pltpu.sync_copy(idx_hbm.at[pl.ds(cbase * CHUNK, n_per_w * CHUNK)], idx_v)

        def gather_start(j, b):
            pltpu.async_copy(
                table_hbm.at[idx_v.at[pl.ds(j * CHUNK, CHUNK)]], rows[b], gs[b])

        def gather_wait(j, b):
            pltpu.make_async_copy(
                table_hbm.at[idx_v.at[pl.ds(j * CHUNK, CHUNK)]], rows[b],
                gs[b]).wait()

        def write_start(j, b):
            pltpu.async_copy(
                rows[b], out_hbm.at[pl.ds((cbase + j) * CHUNK, CHUNK)], ws[b])

        def write_wait(j, b):
            pltpu.make_async_copy(
                rows[b], out_hbm.at[pl.ds((cbase + j) * CHUNK, CHUNK)],
                ws[b]).wait()

        for b in range(LOOK):
            gather_start(b, b)

        def body(g, carry):
            for b in range(NBUF):
                j = g * NBUF + b
                bp = (b + LOOK) % NBUF
                gather_wait(j, b)
                write_start(j, b)

                @pl.when(j + LOOK < n_per_w)
                def _():
                    @pl.when(j + LOOK >= NBUF)
                    def _w():
                        write_wait(j + LOOK - NBUF, bp)

                    gather_start(j + LOOK, bp)

            return carry

        lax.fori_loop(0, n_per_w // NBUF, body, 0)

        for b in range(NBUF):
            write_wait(n_per_w - NBUF + b, b)

    return k


def kernel(input, table):
    batch, hist = input.shape
    flat_t = jnp.transpose(input).reshape(-1).astype(jnp.int32)
    n_chunks = flat_t.shape[0] // CHUNK
    out = _gather_fn(n_chunks // NW)(flat_t, table)
    out = out.reshape(hist, batch, DIM)
    return jnp.transpose(out, (1, 0, 2))
